# baseline (speedup 1.0000x reference)
import functools
import math

import jax
import jax.numpy as jnp
from jax import lax
from jax.experimental import pallas as pl
from jax.experimental.pallas import tpu as pltpu
from jax.experimental.pallas import tpu_sc as plsc

N = 10000
E = 320000
DIM = 128
NSEG = 64
NLAYERS = 4

NC = 2
NS = 16
NW = NC * NS
CHUNK = 128
BB = 4
NBLK = 20
NCHUNK = NBLK * BB
EPW = NCHUNK * CHUNK
E_PAD = NW * EPW
NPAD = 10112
ROWS_PER_TILE = NPAD // NS

BLK = 2000
GRID = N // BLK

_BN_RSQRT = 1.0 / math.sqrt(1.0 + 1e-5)


def _agg_body(h_hbm, src_hbm, dst_hbm, out_hbm, srcA, dstA, srcB, dstB,
              buf0, buf1, aggsh, gsem, ssem0, ssem1, isem_s, isem_d):
    c = lax.axis_index("c")
    s = lax.axis_index("s")
    wid = c * NS + s
    bufs = (buf0, buf1)
    gsems = (gsem, gsem)
    ssems = (ssem0, ssem1)
    slots = ((srcA, dstA), (srcB, dstB))

    def zelem(t, carry):
        buf0[t // 8, pl.ds((t % 8) * 16, 16)] = jnp.zeros((16,), jnp.float32)
        return carry

    lax.fori_loop(0, CHUNK * 8, zelem, 0)
    base = s * ROWS_PER_TILE
    for k in range(4):
        pltpu.sync_copy(buf0, aggsh.at[pl.ds(base + k * CHUNK, CHUNK)])
    pltpu.sync_copy(buf0.at[pl.ds(0, 120)],
                    aggsh.at[pl.ds(base + 4 * CHUNK, 120)])
    plsc.subcore_barrier()

    def g_start(sr, j, p):
        pltpu.async_copy(h_hbm.at[sr.at[j]], bufs[p], gsems[p])

    def g_wait(sr, j, p):
        pltpu.make_async_copy(h_hbm.at[sr.at[j]], bufs[p], gsems[p]).wait()

    def s_start(dr, j, p):
        pltpu.async_copy(bufs[p], aggsh.at[dr.at[j]], ssems[p], add=True)

    def s_wait(dr, j, p):
        pltpu.make_async_copy(bufs[p], aggsh.at[dr.at[j]], ssems[p]).wait()

    def idx_start(b, slot):
        pltpu.async_copy(src_hbm.at[wid, b], slots[slot][0], isem_s)
        pltpu.async_copy(dst_hbm.at[wid, b], slots[slot][1], isem_d)

    def idx_wait(b, slot):
        pltpu.make_async_copy(src_hbm.at[wid, b], slots[slot][0], isem_s).wait()
        pltpu.make_async_copy(dst_hbm.at[wid, b], slots[slot][1], isem_d).wait()

    def do_block(b, slot, first=False, last=False):
        sr, dr = slots[slot]
        prev = 1 - slot
        for j in range(BB):
            p = j % 2
            g_wait(sr, j, p)
            s_start(dr, j, p)
            if j == 0:
                if not first:
                    s_wait(slots[prev][1], BB - 1, 1)
                    if not last:
                        idx_start(b + 1, prev)
            else:
                s_wait(dr, j - 1, 1 - p)
            if j < BB - 1:
                g_start(sr, j + 1, 1 - p)
            elif not last:
                idx_wait(b + 1, prev)
                g_start(slots[prev][0], 0, 1 - p)

    pltpu.sync_copy(src_hbm.at[wid, 0], srcA)
    pltpu.sync_copy(dst_hbm.at[wid, 0], dstA)
    idx_start(1, 1)
    g_start(srcA, 0, 0)
    do_block(0, 0, first=True)

    def pair_body(bb, carry):
        do_block(2 * bb + 1, 1)
        do_block(2 * bb + 2, 0)
        return carry

    lax.fori_loop(0, (NBLK - 4) // 2, pair_body, 0)
    do_block(NBLK - 3, 1)
    do_block(NBLK - 2, 0)
    do_block(NBLK - 1, 1, last=True)
    s_wait(slots[(NBLK - 1) % 2][1], BB - 1, 1)
    plsc.subcore_barrier()

    pltpu.sync_copy(
        aggsh.at[pl.ds(s * ROWS_PER_TILE, ROWS_PER_TILE)],
        out_hbm.at[c, pl.ds(s * ROWS_PER_TILE, ROWS_PER_TILE)],
    )


@functools.cache
def _make_agg():
    return pl.kernel(
        _agg_body,
        mesh=plsc.VectorSubcoreMesh(core_axis_name="c", subcore_axis_name="s"),
        out_type=jax.ShapeDtypeStruct((NC, NPAD, DIM), jnp.float32),
        scratch_types=[
            pltpu.VMEM((BB, CHUNK), jnp.int32),
            pltpu.VMEM((BB, CHUNK), jnp.int32),
            pltpu.VMEM((BB, CHUNK), jnp.int32),
            pltpu.VMEM((BB, CHUNK), jnp.int32),
            pltpu.VMEM((CHUNK, DIM), jnp.float32),
            pltpu.VMEM((CHUNK, DIM), jnp.float32),
            pltpu.VMEM_SHARED((NPAD, DIM), jnp.float32),
        ] + [pltpu.SemaphoreType.DMA] * 5,
    )


def _mlp_math(eps_ref, h_ref, a0_ref, a1_ref, w1_ref, b1_ref, w2_ref, b2_ref,
              g_ref, bb_ref):
    h = h_ref[...]
    z = (1.0 + eps_ref[0, 0]) * h + a0_ref[...] + a1_ref[...]
    z = jnp.maximum(
        jnp.dot(z, w1_ref[...], preferred_element_type=jnp.float32) + b1_ref[...],
        0.0)
    z = jnp.dot(z, w2_ref[...], preferred_element_type=jnp.float32) + b2_ref[...]
    z = g_ref[...] * (z * _BN_RSQRT) + bb_ref[...]
    return h, jnp.maximum(z, 0.0)


def _onehot(batch_ref):
    seg = lax.broadcasted_iota(jnp.int32, (NSEG, BLK), 0)
    return (seg == batch_ref[0]).astype(jnp.float32)


def _mlp_body(eps_ref, h_ref, a0_ref, a1_ref, w1_ref, b1_ref, w2_ref, b2_ref,
              g_ref, bb_ref, batch_ref, hout_ref, pool_ref):
    h, h1 = _mlp_math(eps_ref, h_ref, a0_ref, a1_ref, w1_ref, b1_ref, w2_ref,
                      b2_ref, g_ref, bb_ref)
    hout_ref[...] = h1
    oh = _onehot(batch_ref)

    @pl.when(pl.program_id(0) == 0)
    def _():
        pool_ref[...] = jnp.zeros_like(pool_ref)

    pool_ref[...] += jnp.dot(oh, h1, preferred_element_type=jnp.float32)


def _mlp_body_poolin(eps_ref, h_ref, a0_ref, a1_ref, w1_ref, b1_ref, w2_ref,
                     b2_ref, g_ref, bb_ref, batch_ref, hout_ref, pool_ref,
                     poolx_ref):
    h, h1 = _mlp_math(eps_ref, h_ref, a0_ref, a1_ref, w1_ref, b1_ref, w2_ref,
                      b2_ref, g_ref, bb_ref)
    hout_ref[...] = h1
    oh = _onehot(batch_ref)

    @pl.when(pl.program_id(0) == 0)
    def _():
        pool_ref[...] = jnp.zeros_like(pool_ref)
        poolx_ref[...] = jnp.zeros_like(poolx_ref)

    pool_ref[...] += jnp.dot(oh, h1, preferred_element_type=jnp.float32)
    poolx_ref[...] += jnp.dot(oh, h, preferred_element_type=jnp.float32)


def _mlp_body_final(eps_ref, h_ref, a0_ref, a1_ref, w1_ref, b1_ref, w2_ref,
                    b2_ref, g_ref, bb_ref, batch_ref, p0_ref, p1_ref, p2_ref,
                    p3_ref, wp0_ref, wp1_ref, wp2_ref, wp3_ref, wp4_ref,
                    bp_ref, out_ref):
    _, h1 = _mlp_math(eps_ref, h_ref, a0_ref, a1_ref, w1_ref, b1_ref, w2_ref,
                      b2_ref, g_ref, bb_ref)
    oh = _onehot(batch_ref)
    p4 = jnp.dot(oh, h1, preferred_element_type=jnp.float32)

    @pl.when(pl.program_id(0) == 0)
    def _():
        acc = bp_ref[...]
        acc = acc + jnp.dot(p0_ref[...], wp0_ref[...],
                            preferred_element_type=jnp.float32)
        acc = acc + jnp.dot(p1_ref[...], wp1_ref[...],
                            preferred_element_type=jnp.float32)
        acc = acc + jnp.dot(p2_ref[...], wp2_ref[...],
                            preferred_element_type=jnp.float32)
        acc = acc + jnp.dot(p3_ref[...], wp3_ref[...],
                            preferred_element_type=jnp.float32)
        out_ref[...] = acc

    out_ref[...] += jnp.dot(p4, wp4_ref[...],
                            preferred_element_type=jnp.float32)


def _row_spec():
    return pl.BlockSpec((BLK, DIM), lambda i: (i, 0))


def _full_spec(shape):
    nd = len(shape)
    return pl.BlockSpec(shape, lambda i: (0,) * nd)


_MLP_IN_SPECS = [
    pl.BlockSpec(memory_space=pltpu.SMEM),
    _row_spec(),
    _row_spec(),
    _row_spec(),
    _full_spec((DIM, DIM)),
    _full_spec((1, DIM)),
    _full_spec((DIM, DIM)),
    _full_spec((1, DIM)),
    _full_spec((1, DIM)),
    _full_spec((1, DIM)),
    pl.BlockSpec((1, 1, BLK), lambda i: (i, 0, 0)),
]

_mlp_call = pl.pallas_call(
    _mlp_body,
    grid=(GRID,),
    in_specs=_MLP_IN_SPECS,
    out_specs=[_row_spec(), _full_spec((NSEG, DIM))],
    out_shape=[
        jax.ShapeDtypeStruct((N, DIM), jnp.float32),
        jax.ShapeDtypeStruct((NSEG, DIM), jnp.float32),
    ],
)

_mlp_call_poolin = pl.pallas_call(
    _mlp_body_poolin,
    grid=(GRID,),
    in_specs=_MLP_IN_SPECS,
    out_specs=[_row_spec(), _full_spec((NSEG, DIM)), _full_spec((NSEG, DIM))],
    out_shape=[
        jax.ShapeDtypeStruct((N, DIM), jnp.float32),
        jax.ShapeDtypeStruct((NSEG, DIM), jnp.float32),
        jax.ShapeDtypeStruct((NSEG, DIM), jnp.float32),
    ],
)


_mlp_call_final = pl.pallas_call(
    _mlp_body_final,
    grid=(GRID,),
    in_specs=_MLP_IN_SPECS + [
        _full_spec((NSEG, DIM)),
        _full_spec((NSEG, DIM)),
        _full_spec((NSEG, DIM)),
        _full_spec((NSEG, DIM)),
        _full_spec((DIM, DIM)),
        _full_spec((DIM, DIM)),
        _full_spec((DIM, DIM)),
        _full_spec((DIM, DIM)),
        _full_spec((DIM, DIM)),
        _full_spec((1, DIM)),
    ],
    out_specs=_full_spec((NSEG, DIM)),
    out_shape=jax.ShapeDtypeStruct((NSEG, DIM), jnp.float32),
)


def kernel(x, params, edge_index, batch):
    pad = E_PAD - E
    src2d = jnp.concatenate(
        [edge_index[0], jnp.zeros((pad,), jnp.int32)]).reshape(
            NW, NBLK, BB, CHUNK)
    dst2d = jnp.concatenate(
        [edge_index[1], jnp.full((pad,), NPAD - 1, jnp.int32)]).reshape(
            NW, NBLK, BB, CHUNK)
    batch3d = batch.reshape(GRID, 1, BLK)

    h = x
    pools = []
    for l in range(NLAYERS):
        agg = _make_agg()(h, src2d, dst2d)[:, :N, :]
        args = (
            params["eps_%d" % l].reshape(1, 1),
            h, agg[0], agg[1],
            params["W1_%d" % l], params["b1_%d" % l].reshape(1, DIM),
            params["W2_%d" % l], params["b2_%d" % l].reshape(1, DIM),
            params["bn_g_%d" % l].reshape(1, DIM),
            params["bn_b_%d" % l].reshape(1, DIM),
            batch3d,
        )
        if l == 0:
            h, p, px = _mlp_call_poolin(*args)
            pools = [px, p]
        elif l < NLAYERS - 1:
            h, p = _mlp_call(*args)
            pools.append(p)
        else:
            wp = params["W_pred"]
            wslices = [wp[k * DIM:(k + 1) * DIM] for k in range(NLAYERS + 1)]
            return _mlp_call_final(
                *args, *pools, *wslices,
                params["b_pred"].reshape(1, DIM))


# speedup vs baseline: 3.1736x; 3.1736x over previous
import functools
import math

import jax
import jax.numpy as jnp
from jax import lax
from jax.experimental import pallas as pl
from jax.experimental.pallas import tpu as pltpu
from jax.experimental.pallas import tpu_sc as plsc

N = 10000
E = 320000
DIM = 128
NSEG = 64
NLAYERS = 4

NC = 2
NS = 16
NW = NC * NS
CHUNK = 100
BB = 4
NBLK = 25
NCHUNK = NBLK * BB
EPW = NCHUNK * CHUNK
E_PAD = NW * EPW
NPAD = 10112
ROWS_PER_TILE = NPAD // NS

BLK = 2000
GRID = N // BLK

_BN_RSQRT = 1.0 / math.sqrt(1.0 + 1e-5)


def _agg_body(h_hbm, src_hbm, dst_hbm, out_hbm, srcA, dstA, srcB, dstB,
              buf0, buf1, aggsh, gsem, ssem0, ssem1, isem_s, isem_d):
    c = lax.axis_index("c")
    s = lax.axis_index("s")
    wid = c * NS + s
    bufs = (buf0, buf1)
    gsems = (gsem, gsem)
    ssems = (ssem0, ssem1)
    slots = ((srcA, dstA), (srcB, dstB))

    def zelem(t, carry):
        buf0[t // 8, pl.ds((t % 8) * 16, 16)] = jnp.zeros((16,), jnp.float32)
        return carry

    lax.fori_loop(0, CHUNK * 8, zelem, 0)
    base = s * ROWS_PER_TILE
    for k in range(6):
        pltpu.sync_copy(buf0, aggsh.at[pl.ds(base + k * CHUNK, CHUNK)])
    pltpu.sync_copy(buf0.at[pl.ds(0, 32)],
                    aggsh.at[pl.ds(base + 6 * CHUNK, 32)])
    plsc.subcore_barrier()

    def g_start(sr, j, p):
        pltpu.async_copy(h_hbm.at[sr.at[j]], bufs[p], gsems[p])

    def g_wait(sr, j, p):
        pltpu.make_async_copy(h_hbm.at[sr.at[j]], bufs[p], gsems[p]).wait()

    def s_start(dr, j, p):
        pltpu.async_copy(bufs[p], aggsh.at[dr.at[j]], ssems[p], add=True)

    def s_wait(dr, j, p):
        pltpu.make_async_copy(bufs[p], aggsh.at[dr.at[j]], ssems[p]).wait()

    def idx_start(b, slot):
        pltpu.async_copy(src_hbm.at[wid, b], slots[slot][0], isem_s)
        pltpu.async_copy(dst_hbm.at[wid, b], slots[slot][1], isem_d)

    def idx_wait(b, slot):
        pltpu.make_async_copy(src_hbm.at[wid, b], slots[slot][0], isem_s).wait()
        pltpu.make_async_copy(dst_hbm.at[wid, b], slots[slot][1], isem_d).wait()

    def do_block(b, slot, first=False, last=False):
        sr, dr = slots[slot]
        prev = 1 - slot
        for j in range(BB):
            p = j % 2
            g_wait(sr, j, p)
            s_start(dr, j, p)
            if j == 0:
                if not first:
                    s_wait(slots[prev][1], BB - 1, 1)
                    if not last:
                        idx_start(b + 1, prev)
            else:
                s_wait(dr, j - 1, 1 - p)
            if j < BB - 1:
                g_start(sr, j + 1, 1 - p)
            elif not last:
                idx_wait(b + 1, prev)
                g_start(slots[prev][0], 0, 1 - p)

    pltpu.sync_copy(src_hbm.at[wid, 0], srcA)
    pltpu.sync_copy(dst_hbm.at[wid, 0], dstA)
    idx_start(1, 1)
    g_start(srcA, 0, 0)
    do_block(0, 0, first=True)

    def pair_body(bb, carry):
        do_block(2 * bb + 1, 1)
        do_block(2 * bb + 2, 0)
        return carry

    lax.fori_loop(0, (NBLK - 3) // 2, pair_body, 0)
    do_block(NBLK - 2, 1)
    do_block(NBLK - 1, 0, last=True)
    s_wait(slots[(NBLK - 1) % 2][1], BB - 1, 1)
    plsc.subcore_barrier()

    pltpu.sync_copy(
        aggsh.at[pl.ds(s * ROWS_PER_TILE, ROWS_PER_TILE)],
        out_hbm.at[c, pl.ds(s * ROWS_PER_TILE, ROWS_PER_TILE)],
    )


@functools.cache
def _make_agg():
    return pl.kernel(
        _agg_body,
        mesh=plsc.VectorSubcoreMesh(core_axis_name="c", subcore_axis_name="s"),
        out_type=jax.ShapeDtypeStruct((NC, NPAD, DIM), jnp.float32),
        scratch_types=[
            pltpu.VMEM((BB, CHUNK), jnp.int32),
            pltpu.VMEM((BB, CHUNK), jnp.int32),
            pltpu.VMEM((BB, CHUNK), jnp.int32),
            pltpu.VMEM((BB, CHUNK), jnp.int32),
            pltpu.VMEM((CHUNK, DIM), jnp.float32),
            pltpu.VMEM((CHUNK, DIM), jnp.float32),
            pltpu.VMEM_SHARED((NPAD, DIM), jnp.float32),
        ] + [pltpu.SemaphoreType.DMA] * 5,
    )


def _mlp_math(eps_ref, h_ref, a0_ref, a1_ref, w1_ref, b1_ref, w2_ref, b2_ref,
              g_ref, bb_ref):
    h = h_ref[...]
    z = (1.0 + eps_ref[0, 0]) * h + a0_ref[...] + a1_ref[...]
    z = jnp.maximum(
        jnp.dot(z, w1_ref[...], preferred_element_type=jnp.float32) + b1_ref[...],
        0.0)
    z = jnp.dot(z, w2_ref[...], preferred_element_type=jnp.float32) + b2_ref[...]
    z = g_ref[...] * (z * _BN_RSQRT) + bb_ref[...]
    return h, jnp.maximum(z, 0.0)


def _onehot(batch_ref):
    seg = lax.broadcasted_iota(jnp.int32, (NSEG, BLK), 0)
    return (seg == batch_ref[0]).astype(jnp.float32)


def _mlp_body(eps_ref, h_ref, a0_ref, a1_ref, w1_ref, b1_ref, w2_ref, b2_ref,
              g_ref, bb_ref, batch_ref, hout_ref, pool_ref):
    h, h1 = _mlp_math(eps_ref, h_ref, a0_ref, a1_ref, w1_ref, b1_ref, w2_ref,
                      b2_ref, g_ref, bb_ref)
    hout_ref[...] = h1
    oh = _onehot(batch_ref)

    @pl.when(pl.program_id(0) == 0)
    def _():
        pool_ref[...] = jnp.zeros_like(pool_ref)

    pool_ref[...] += jnp.dot(oh, h1, preferred_element_type=jnp.float32)


def _mlp_body_poolin(eps_ref, h_ref, a0_ref, a1_ref, w1_ref, b1_ref, w2_ref,
                     b2_ref, g_ref, bb_ref, batch_ref, hout_ref, pool_ref,
                     poolx_ref):
    h, h1 = _mlp_math(eps_ref, h_ref, a0_ref, a1_ref, w1_ref, b1_ref, w2_ref,
                      b2_ref, g_ref, bb_ref)
    hout_ref[...] = h1
    oh = _onehot(batch_ref)

    @pl.when(pl.program_id(0) == 0)
    def _():
        pool_ref[...] = jnp.zeros_like(pool_ref)
        poolx_ref[...] = jnp.zeros_like(poolx_ref)

    pool_ref[...] += jnp.dot(oh, h1, preferred_element_type=jnp.float32)
    poolx_ref[...] += jnp.dot(oh, h, preferred_element_type=jnp.float32)


def _mlp_body_final(eps_ref, h_ref, a0_ref, a1_ref, w1_ref, b1_ref, w2_ref,
                    b2_ref, g_ref, bb_ref, batch_ref, p0_ref, p1_ref, p2_ref,
                    p3_ref, wp0_ref, wp1_ref, wp2_ref, wp3_ref, wp4_ref,
                    bp_ref, out_ref):
    _, h1 = _mlp_math(eps_ref, h_ref, a0_ref, a1_ref, w1_ref, b1_ref, w2_ref,
                      b2_ref, g_ref, bb_ref)
    oh = _onehot(batch_ref)
    p4 = jnp.dot(oh, h1, preferred_element_type=jnp.float32)

    @pl.when(pl.program_id(0) == 0)
    def _():
        acc = bp_ref[...]
        acc = acc + jnp.dot(p0_ref[...], wp0_ref[...],
                            preferred_element_type=jnp.float32)
        acc = acc + jnp.dot(p1_ref[...], wp1_ref[...],
                            preferred_element_type=jnp.float32)
        acc = acc + jnp.dot(p2_ref[...], wp2_ref[...],
                            preferred_element_type=jnp.float32)
        acc = acc + jnp.dot(p3_ref[...], wp3_ref[...],
                            preferred_element_type=jnp.float32)
        out_ref[...] = acc

    out_ref[...] += jnp.dot(p4, wp4_ref[...],
                            preferred_element_type=jnp.float32)


def _row_spec():
    return pl.BlockSpec((BLK, DIM), lambda i: (i, 0))


def _full_spec(shape):
    nd = len(shape)
    return pl.BlockSpec(shape, lambda i: (0,) * nd)


_MLP_IN_SPECS = [
    pl.BlockSpec(memory_space=pltpu.SMEM),
    _row_spec(),
    _row_spec(),
    _row_spec(),
    _full_spec((DIM, DIM)),
    _full_spec((1, DIM)),
    _full_spec((DIM, DIM)),
    _full_spec((1, DIM)),
    _full_spec((1, DIM)),
    _full_spec((1, DIM)),
    pl.BlockSpec((1, 1, BLK), lambda i: (i, 0, 0)),
]

_mlp_call = pl.pallas_call(
    _mlp_body,
    grid=(GRID,),
    in_specs=_MLP_IN_SPECS,
    out_specs=[_row_spec(), _full_spec((NSEG, DIM))],
    out_shape=[
        jax.ShapeDtypeStruct((N, DIM), jnp.float32),
        jax.ShapeDtypeStruct((NSEG, DIM), jnp.float32),
    ],
)

_mlp_call_poolin = pl.pallas_call(
    _mlp_body_poolin,
    grid=(GRID,),
    in_specs=_MLP_IN_SPECS,
    out_specs=[_row_spec(), _full_spec((NSEG, DIM)), _full_spec((NSEG, DIM))],
    out_shape=[
        jax.ShapeDtypeStruct((N, DIM), jnp.float32),
        jax.ShapeDtypeStruct((NSEG, DIM), jnp.float32),
        jax.ShapeDtypeStruct((NSEG, DIM), jnp.float32),
    ],
)


_mlp_call_final = pl.pallas_call(
    _mlp_body_final,
    grid=(GRID,),
    in_specs=_MLP_IN_SPECS + [
        _full_spec((NSEG, DIM)),
        _full_spec((NSEG, DIM)),
        _full_spec((NSEG, DIM)),
        _full_spec((NSEG, DIM)),
        _full_spec((DIM, DIM)),
        _full_spec((DIM, DIM)),
        _full_spec((DIM, DIM)),
        _full_spec((DIM, DIM)),
        _full_spec((DIM, DIM)),
        _full_spec((1, DIM)),
    ],
    out_specs=_full_spec((NSEG, DIM)),
    out_shape=jax.ShapeDtypeStruct((NSEG, DIM), jnp.float32),
)


def kernel(x, params, edge_index, batch):
    src2d = edge_index[0].reshape(NW, NBLK, BB, CHUNK)
    dst2d = edge_index[1].reshape(NW, NBLK, BB, CHUNK)
    batch3d = batch.reshape(GRID, 1, BLK)

    h = x
    pools = []
    for l in range(NLAYERS):
        agg = _make_agg()(h, src2d, dst2d)[:, :N, :]
        args = (
            params["eps_%d" % l].reshape(1, 1),
            h, agg[0], agg[1],
            params["W1_%d" % l], params["b1_%d" % l].reshape(1, DIM),
            params["W2_%d" % l], params["b2_%d" % l].reshape(1, DIM),
            params["bn_g_%d" % l].reshape(1, DIM),
            params["bn_b_%d" % l].reshape(1, DIM),
            batch3d,
        )
        if l == 0:
            h, p, px = _mlp_call_poolin(*args)
            pools = [px, p]
        elif l < NLAYERS - 1:
            h, p = _mlp_call(*args)
            pools.append(p)
        else:
            wp = params["W_pred"]
            wslices = [wp[k * DIM:(k + 1) * DIM] for k in range(NLAYERS + 1)]
            return _mlp_call_final(
                *args, *pools, *wslices,
                params["b_pred"].reshape(1, DIM))
